# baseline (speedup 1.0000x reference)
import functools

import jax
import jax.numpy as jnp
import numpy as np
from jax.experimental import pallas as pl
from jax.experimental.pallas import tpu as pltpu

N_PTS_ = 128
MIN_DEPTH_ = 2.0
MAX_DEPTH_ = 6.0
BM = 8192


def _body(odt_ref, s_ref, z_ref, pts_ref, len_ref):
    odt = odt_ref[...]
    s = s_ref[...]
    flat = jax.lax.dot_general(
        odt, s, (((0,), (0,)), ((), ())),
        preferred_element_type=jnp.float32,
        precision=jax.lax.Precision.DEFAULT)
    for c in range(3):
        pts_ref[c, :, :] = flat[:, c * N_PTS_:(c + 1) * N_PTS_]
    len_ref[...] = jnp.broadcast_to(z_ref[...], (odt.shape[1], N_PTS_))


@functools.partial(jax.jit, static_argnums=())
def kernel(origins, directions):
    m = origins.shape[0]
    step = (MAX_DEPTH_ - MIN_DEPTH_) / N_PTS_
    z = np.arange(MIN_DEPTH_, MAX_DEPTH_, step, dtype=np.float32)
    s = np.zeros((6, 3 * N_PTS_), dtype=np.float32)
    for c in range(3):
        s[c, c * N_PTS_:(c + 1) * N_PTS_] = 1.0
        s[3 + c, c * N_PTS_:(c + 1) * N_PTS_] = z
    s = jnp.asarray(s)
    zrow = jnp.asarray(z[None, :])

    odt = jnp.concatenate([origins.T, directions.T], axis=0)

    grid = (m // BM,)
    pts_t, lens = pl.pallas_call(
        _body,
        grid=grid,
        in_specs=[
            pl.BlockSpec((6, BM), lambda i: (0, i)),
            pl.BlockSpec((6, 3 * N_PTS_), lambda i: (0, 0)),
            pl.BlockSpec((1, N_PTS_), lambda i: (0, 0)),
        ],
        out_specs=[
            pl.BlockSpec((3, BM, N_PTS_), lambda i: (0, i, 0)),
            pl.BlockSpec((BM, N_PTS_), lambda i: (i, 0)),
        ],
        out_shape=[
            jax.ShapeDtypeStruct((3, m, N_PTS_), jnp.float32),
            jax.ShapeDtypeStruct((m, N_PTS_), jnp.float32),
        ],
        compiler_params=pltpu.CompilerParams(
            dimension_semantics=("parallel",)),
    )(odt, s, zrow)

    return (jnp.transpose(pts_t, (1, 2, 0)), lens.reshape(m, N_PTS_, 1))
